# baseline (speedup 1.0000x reference)
import functools

import jax
import jax.numpy as jnp
from jax.experimental import pallas as pl
from jax.experimental.pallas import tpu as pltpu
from jax.experimental.pallas import tpu_sc as plsc

_L = 16


@functools.partial(
    pl.kernel,
    out_type=jax.ShapeDtypeStruct((_L,), jnp.float32),
    mesh=plsc.VectorSubcoreMesh(
        core_axis_name="c", subcore_axis_name="s", num_cores=1, num_subcores=1
    ),
    compiler_params=pltpu.CompilerParams(needs_layout_passes=False),
    scratch_types=[
        pltpu.VMEM((3 * _L,), jnp.float32),
        pltpu.VMEM((_L,), jnp.float32),
        pltpu.SemaphoreType.DMA,
    ],
)
def _sc_sample(packed_hbm, out_hbm, packed_v, out_v, sem):
    pltpu.async_copy(packed_hbm, packed_v, sem).wait()

    p = packed_v[pl.ds(_L, _L)]
    total = jnp.sum(p)
    cdf = plsc.cumsum(p / total)
    idx = plsc.all_reduce_population_count(cdf < packed_v[pl.ds(2 * _L, _L)])
    idx = jnp.minimum(idx, _L - 1)
    out_v[...] = plsc.load_gather(packed_v.at[pl.ds(0, _L)], [idx])

    pltpu.sync_copy(out_v, out_hbm)


@jax.jit
def kernel(values, probs, u):
    packed = jnp.concatenate([values, probs, jnp.broadcast_to(u, (_L,))])
    return _sc_sample(packed)[:1]
